# baseline (speedup 1.0000x reference)
import functools

import jax
import jax.numpy as jnp
from jax import lax
from jax.experimental import pallas as pl
from jax.experimental.pallas import tpu as pltpu
from jax.experimental.pallas import tpu_sc as plsc

S = 200
B = 4096
D = 64
V = 1000000
V_MAIN = 999936
NC, NS = 2, 16
NW = NC * NS
D_PER_SC = D // NC
S_BLK = 25
B_BLK = 2048
N_TOK = S_BLK * B_BLK
N_STAGE = 10
SEG = V // N_STAGE
ROWS_PER_PIECE = 4
PIECE = ROWS_PER_PIECE * B_BLK
PIECES = [ROWS_PER_PIECE] * 6 + [1]

DT_CHUNKS = [325 * 128] * 24 + [12 * 128]
DT_BUF = 325 * 128


def _detile(w_t, w_tail):
    mesh = plsc.VectorSubcoreMesh(core_axis_name="c", subcore_axis_name="s")

    @functools.partial(
        pl.kernel,
        out_type=jax.ShapeDtypeStruct((D * V,), jnp.float32),
        mesh=mesh,
        scratch_types=[
            pltpu.VMEM((DT_BUF,), jnp.float32),
            pltpu.VMEM((DT_BUF,), jnp.float32),
            pltpu.SemaphoreType.DMA,
            pltpu.SemaphoreType.DMA,
            pltpu.SemaphoreType.DMA,
        ],
    )
    def body(w_hbm, wt_hbm, wf_hbm, buf0, buf1, sem_r, sw0, sw1):
        bufs = (buf0, buf1)
        cid = lax.axis_index("c")
        sid = lax.axis_index("s")
        wid = sid * NC + cid
        wsems = (sw0, sw1)
        for r in range(D // NW):
            d = wid * (D // NW) + r
            reads = []
            off = 0
            for n in DT_CHUNKS:
                reads.append((off, n))
                off += n
            pending = {0: None, 1: None}
            h_read = pltpu.async_copy(
                w_hbm.at[d, pl.ds(0, DT_CHUNKS[0])],
                buf0.at[pl.ds(0, DT_CHUNKS[0])],
                sem_r,
            )
            for i, (off, n) in enumerate(reads):
                slot = i % 2
                h_read.wait()
                if i + 1 < len(reads):
                    off2, n2 = reads[i + 1]
                    nslot = (i + 1) % 2
                    if pending[nslot] is not None:
                        pending[nslot].wait()
                        pending[nslot] = None
                    h_read = pltpu.async_copy(
                        w_hbm.at[d, pl.ds(off2, n2)],
                        bufs[nslot].at[pl.ds(0, n2)],
                        sem_r,
                    )
                pending[slot] = pltpu.async_copy(
                    bufs[slot].at[pl.ds(0, n)],
                    wf_hbm.at[pl.ds(d * V + off, n)],
                    wsems[slot],
                )
            for slot in (0, 1):
                if pending[slot] is not None:
                    pending[slot].wait()
                    pending[slot] = None
            pltpu.sync_copy(wt_hbm.at[d, pl.ds(0, 64)], buf0.at[pl.ds(0, 64)])
            pltpu.sync_copy(buf0.at[pl.ds(0, 64)],
                            wf_hbm.at[pl.ds(d * V + V_MAIN, 64)])

    return body(w_t, w_tail)


def _lookup(w_feat, ids_t):
    mesh = plsc.VectorSubcoreMesh(core_axis_name="c", subcore_axis_name="s")

    @functools.partial(
        pl.kernel,
        out_type=jax.ShapeDtypeStruct((S, D, B), jnp.float32),
        mesh=mesh,
        scratch_types=[
            pltpu.VMEM_SHARED((V,), jnp.float32),
            pltpu.VMEM((N_TOK,), jnp.int32),
            pltpu.VMEM((2, PIECE), jnp.float32),
            pltpu.SemaphoreType.DMA,
            pltpu.SemaphoreType.DMA,
            pltpu.SemaphoreType.DMA,
            pltpu.SemaphoreType.DMA,
        ],
        compiler_params=pltpu.CompilerParams(use_tc_tiling_on_sc=False),
    )
    def body(wf_hbm, ids_hbm, out_hbm, row_sh, ids_v, buf, sg0, sg1, sw0, sw1):
        gsems = (sg0, sg1)
        cid = lax.axis_index("c")
        sid = lax.axis_index("s")
        s0 = (sid // 2) * S_BLK
        b0 = (sid % 2) * B_BLK
        for s in range(S_BLK):
            pltpu.sync_copy(
                ids_hbm.at[s0 + s, pl.ds(b0, B_BLK)],
                ids_v.at[pl.ds(s * B_BLK, B_BLK)],
            )
        wsems = (sw0, sw1)

        def d_body(i, carry):
            d = cid * D_PER_SC + i

            @pl.when(sid < N_STAGE)
            def _stage():
                pltpu.sync_copy(
                    wf_hbm.at[pl.ds(d * V + sid * SEG, SEG)],
                    row_sh.at[pl.ds(sid * SEG, SEG)],
                )

            plsc.subcore_barrier()
            rows_at = [0]
            for nrows in PIECES:
                rows_at.append(rows_at[-1] + nrows)

            def issue_gather(p):
                n = PIECES[p] * B_BLK
                return pltpu.async_copy(
                    row_sh.at[ids_v.at[pl.ds(rows_at[p] * B_BLK, n)]],
                    buf.at[p % 2, pl.ds(0, n)],
                    gsems[p % 2],
                )

            gathers = {0: issue_gather(0)}
            w_prev = []
            for p, nrows in enumerate(PIECES):
                slot = p % 2
                for h in w_prev:
                    h.wait()
                if p + 1 < len(PIECES):
                    gathers[p + 1] = issue_gather(p + 1)
                gathers.pop(p).wait()
                w_prev = [
                    pltpu.async_copy(
                        buf.at[slot, pl.ds(r * B_BLK, B_BLK)],
                        out_hbm.at[s0 + rows_at[p] + r, d, pl.ds(b0, B_BLK)],
                        wsems[slot],
                    )
                    for r in range(nrows)
                ]
            for h in w_prev:
                h.wait()
            plsc.subcore_barrier()
            return carry

        lax.fori_loop(0, D_PER_SC, d_body, 0)

    return body(w_feat, ids_t)


def kernel(token_ids, weight):
    ids_t = token_ids.T.astype(jnp.int32)
    w_t = weight.T
    w_tail = jnp.concatenate(
        [w_t[:, V_MAIN:], jnp.zeros((D, 128 - (V - V_MAIN)), jnp.float32)], axis=1
    )
    w_feat = _detile(w_t, w_tail)
    out_phys = _lookup(w_feat, ids_t)
    return out_phys.transpose(2, 0, 1)


# speedup vs baseline: 1.0014x; 1.0014x over previous
import functools

import jax
import jax.numpy as jnp
from jax import lax
from jax.experimental import pallas as pl
from jax.experimental.pallas import tpu as pltpu
from jax.experimental.pallas import tpu_sc as plsc

S = 200
B = 4096
D = 64
V = 1000000
V_MAIN = 999936
NC, NS = 2, 16
NW = NC * NS
D_PER_SC = D // NC
S_BLK = 25
B_BLK = 2048
N_TOK = S_BLK * B_BLK
SEG_A = 62504
SEG_LAST = V - 15 * SEG_A
ROWS_PER_PIECE = 4
PIECE = ROWS_PER_PIECE * B_BLK
PIECES = [ROWS_PER_PIECE] * 6 + [1]

DT_CHUNKS = [325 * 128] * 24 + [12 * 128]
DT_BUF = 325 * 128


def _detile(w_t, w_tail):
    mesh = plsc.VectorSubcoreMesh(core_axis_name="c", subcore_axis_name="s")

    @functools.partial(
        pl.kernel,
        out_type=jax.ShapeDtypeStruct((D * V,), jnp.float32),
        mesh=mesh,
        scratch_types=[
            pltpu.VMEM((DT_BUF,), jnp.float32),
            pltpu.VMEM((DT_BUF,), jnp.float32),
            pltpu.SemaphoreType.DMA,
            pltpu.SemaphoreType.DMA,
            pltpu.SemaphoreType.DMA,
        ],
    )
    def body(w_hbm, wt_hbm, wf_hbm, buf0, buf1, sem_r, sw0, sw1):
        bufs = (buf0, buf1)
        cid = lax.axis_index("c")
        sid = lax.axis_index("s")
        wid = sid * NC + cid
        wsems = (sw0, sw1)
        for r in range(D // NW):
            d = wid * (D // NW) + r
            reads = []
            off = 0
            for n in DT_CHUNKS:
                reads.append((off, n))
                off += n
            pending = {0: None, 1: None}
            h_read = pltpu.async_copy(
                w_hbm.at[d, pl.ds(0, DT_CHUNKS[0])],
                buf0.at[pl.ds(0, DT_CHUNKS[0])],
                sem_r,
            )
            for i, (off, n) in enumerate(reads):
                slot = i % 2
                h_read.wait()
                if i + 1 < len(reads):
                    off2, n2 = reads[i + 1]
                    nslot = (i + 1) % 2
                    if pending[nslot] is not None:
                        pending[nslot].wait()
                        pending[nslot] = None
                    h_read = pltpu.async_copy(
                        w_hbm.at[d, pl.ds(off2, n2)],
                        bufs[nslot].at[pl.ds(0, n2)],
                        sem_r,
                    )
                pending[slot] = pltpu.async_copy(
                    bufs[slot].at[pl.ds(0, n)],
                    wf_hbm.at[pl.ds(d * V + off, n)],
                    wsems[slot],
                )
            for slot in (0, 1):
                if pending[slot] is not None:
                    pending[slot].wait()
                    pending[slot] = None
            pltpu.sync_copy(wt_hbm.at[d, pl.ds(0, 64)], buf0.at[pl.ds(0, 64)])
            pltpu.sync_copy(buf0.at[pl.ds(0, 64)],
                            wf_hbm.at[pl.ds(d * V + V_MAIN, 64)])

    return body(w_t, w_tail)


def _lookup(w_feat, ids_t):
    mesh = plsc.VectorSubcoreMesh(core_axis_name="c", subcore_axis_name="s")

    @functools.partial(
        pl.kernel,
        out_type=jax.ShapeDtypeStruct((S, D, B), jnp.float32),
        mesh=mesh,
        scratch_types=[
            pltpu.VMEM_SHARED((V,), jnp.float32),
            pltpu.VMEM((N_TOK,), jnp.int32),
            pltpu.VMEM((2, PIECE), jnp.float32),
            pltpu.SemaphoreType.DMA,
            pltpu.SemaphoreType.DMA,
            pltpu.SemaphoreType.DMA,
            pltpu.SemaphoreType.DMA,
        ],
        compiler_params=pltpu.CompilerParams(use_tc_tiling_on_sc=False),
    )
    def body(wf_hbm, ids_hbm, out_hbm, row_sh, ids_v, buf, sg0, sg1, sw0, sw1):
        gsems = (sg0, sg1)
        cid = lax.axis_index("c")
        sid = lax.axis_index("s")
        s0 = (sid // 2) * S_BLK
        b0 = (sid % 2) * B_BLK
        for s in range(S_BLK):
            pltpu.sync_copy(
                ids_hbm.at[s0 + s, pl.ds(b0, B_BLK)],
                ids_v.at[pl.ds(s * B_BLK, B_BLK)],
            )
        wsems = (sw0, sw1)

        def d_body(i, carry):
            d = cid * D_PER_SC + i

            @pl.when(sid < 15)
            def _stage():
                pltpu.sync_copy(
                    wf_hbm.at[pl.ds(d * V + sid * SEG_A, SEG_A)],
                    row_sh.at[pl.ds(sid * SEG_A, SEG_A)],
                )

            @pl.when(sid == 15)
            def _stage_last():
                pltpu.sync_copy(
                    wf_hbm.at[pl.ds(d * V + 15 * SEG_A, SEG_LAST)],
                    row_sh.at[pl.ds(15 * SEG_A, SEG_LAST)],
                )

            plsc.subcore_barrier()
            rows_at = [0]
            for nrows in PIECES:
                rows_at.append(rows_at[-1] + nrows)

            def issue_gather(p):
                n = PIECES[p] * B_BLK
                return pltpu.async_copy(
                    row_sh.at[ids_v.at[pl.ds(rows_at[p] * B_BLK, n)]],
                    buf.at[p % 2, pl.ds(0, n)],
                    gsems[p % 2],
                )

            gathers = {0: issue_gather(0)}
            w_prev = []
            for p, nrows in enumerate(PIECES):
                slot = p % 2
                for h in w_prev:
                    h.wait()
                if p + 1 < len(PIECES):
                    gathers[p + 1] = issue_gather(p + 1)
                gathers.pop(p).wait()
                w_prev = [
                    pltpu.async_copy(
                        buf.at[slot, pl.ds(r * B_BLK, B_BLK)],
                        out_hbm.at[s0 + rows_at[p] + r, d, pl.ds(b0, B_BLK)],
                        wsems[slot],
                    )
                    for r in range(nrows)
                ]
            for h in w_prev:
                h.wait()
            plsc.subcore_barrier()
            return carry

        lax.fori_loop(0, D_PER_SC, d_body, 0)

    return body(w_feat, ids_t)


def kernel(token_ids, weight):
    ids_t = token_ids.T.astype(jnp.int32)
    w_t = weight.T
    w_tail = jnp.concatenate(
        [w_t[:, V_MAIN:], jnp.zeros((D, 128 - (V - V_MAIN)), jnp.float32)], axis=1
    )
    w_feat = _detile(w_t, w_tail)
    out_phys = _lookup(w_feat, ids_t)
    return out_phys.transpose(2, 0, 1)
